# Initial kernel scaffold; baseline (speedup 1.0000x reference)
import jax
import jax.numpy as jnp
from jax.experimental import pallas as pl


def kernel(seq_a, seq_b):
    raise NotImplementedError("write your pallas kernel here")


# speedup vs baseline: 2.1403x; 2.1403x over previous
import functools

import jax
import jax.numpy as jnp
import numpy as np
from jax import lax
from jax.experimental import pallas as pl
from jax.experimental.pallas import tpu as pltpu
from jax.experimental.pallas import tpu_sc as plsc

_P = 0.5


def _plan_indices(seq_len: int) -> np.ndarray:
    rng = np.random.default_rng(0)
    pa = rng.uniform(0.0, 1.0, size=6)
    idx_list = []
    L = seq_len
    if pa[0] < _P:
        start = int(rng.uniform(0.0, L * 0.1))
        idx = np.arange(start, L, dtype=np.int64)
        idx_list.append(idx)
        L = idx.shape[0]
    if pa[1] < _P:
        end = int(rng.uniform(0.0, L * 0.1))
        idx = np.arange(0, L - end, dtype=np.int64)
        idx_list.append(idx)
        L = idx.shape[0]
    if pa[2] < _P:
        delta = float(np.float16(rng.uniform(0.8, 1.2)))
        idx = np.floor(np.arange(0.0, L, delta)).astype(np.int64)
        idx = np.clip(idx, 0, L - 1)
        idx_list.append(idx)
        L = idx.shape[0]
    if pa[3] < _P:
        margin = int(0.1 * L)
        center = int(rng.uniform(margin, L - margin))
        delta = float(np.float16(rng.uniform(0.5, 1.5)))
        mid = np.arange(center - margin, center + margin, delta).astype(np.int64)
        mid = np.clip(mid, 0, L - 1)
        idx = np.concatenate([
            np.arange(0, center - margin, dtype=np.int64),
            mid,
            np.arange(center + margin, L, dtype=np.int64),
        ])
        idx_list.append(idx)
        L = idx.shape[0]
    if pa[4] < _P:
        margin = int(0.1 * L)
        center = int(rng.uniform(margin, L - margin))
        mid = rng.permutation(np.arange(center - margin, center + margin, dtype=np.int64))
        idx = np.concatenate([
            np.arange(0, center - margin, dtype=np.int64),
            mid,
            np.arange(center + margin, L, dtype=np.int64),
        ])
        idx_list.append(idx)
        L = idx.shape[0]
    if pa[4] < _P:
        margin = int(0.1 * L)
        center = int(rng.uniform(margin, L - margin))
        idx = np.concatenate([
            np.arange(0, center - margin, dtype=np.int64),
            np.arange(center - margin, center + margin, dtype=np.int64)[::-1],
            np.arange(center + margin, L, dtype=np.int64),
        ])
        idx_list.append(idx)
        L = idx.shape[0]
    final = idx_list[0]
    for idx in idx_list[1:]:
        final = final[idx]
    return final.astype(np.int32)


_SEQ_LEN = 4096
_FEAT = 512
_IDX_NP = _plan_indices(_SEQ_LEN)
_OUT_LEN = int(_IDX_NP.shape[0])

_NUM_WORKERS = 32
_PAD_LEN = ((_OUT_LEN + 8 * _NUM_WORKERS - 1) // (8 * _NUM_WORKERS)) * (8 * _NUM_WORKERS)
_ROWS_PER_WORKER = _PAD_LEN // _NUM_WORKERS
assert _ROWS_PER_WORKER <= 128
_IDX_PAD_NP = np.zeros((_PAD_LEN,), dtype=np.int32)
_IDX_PAD_NP[:_OUT_LEN] = _IDX_NP


def _sc_gather(a_hbm, b_hbm, idx_hbm, out_a_hbm, out_b_hbm,
               idx_v, rows_a, rows_b, sem_a, sem_b):
    info = plsc.get_sparse_core_info()
    wid = lax.axis_index("s") * info.num_cores + lax.axis_index("c")
    base = wid * _ROWS_PER_WORKER
    pltpu.sync_copy(idx_hbm.at[pl.ds(base, _ROWS_PER_WORKER)], idx_v)
    cp_a = pltpu.async_copy(a_hbm.at[idx_v], rows_a, sem_a)
    cp_b = pltpu.async_copy(b_hbm.at[idx_v], rows_b, sem_b)
    cp_a.wait()
    pltpu.sync_copy(rows_a, out_a_hbm.at[pl.ds(base, _ROWS_PER_WORKER)])
    cp_b.wait()
    pltpu.sync_copy(rows_b, out_b_hbm.at[pl.ds(base, _ROWS_PER_WORKER)])


@jax.jit
def kernel(seq_a, seq_b):
    idx = jnp.asarray(_IDX_PAD_NP)
    mesh = plsc.VectorSubcoreMesh(core_axis_name="c", subcore_axis_name="s")
    out_a, out_b = pl.kernel(
        _sc_gather,
        mesh=mesh,
        out_type=(
            jax.ShapeDtypeStruct((_PAD_LEN, _FEAT), jnp.float32),
            jax.ShapeDtypeStruct((_PAD_LEN, _FEAT), jnp.float32),
        ),
        scratch_types=[
            pltpu.VMEM((_ROWS_PER_WORKER,), jnp.int32),
            pltpu.VMEM((_ROWS_PER_WORKER, _FEAT), jnp.float32),
            pltpu.VMEM((_ROWS_PER_WORKER, _FEAT), jnp.float32),
            pltpu.SemaphoreType.DMA,
            pltpu.SemaphoreType.DMA,
        ],
    )(seq_a, seq_b, idx)
    return jnp.stack([out_a[:_OUT_LEN], out_b[:_OUT_LEN]], axis=0)


# speedup vs baseline: 2.1788x; 1.0180x over previous
import functools

import jax
import jax.numpy as jnp
import numpy as np
from jax import lax
from jax.experimental import pallas as pl
from jax.experimental.pallas import tpu as pltpu
from jax.experimental.pallas import tpu_sc as plsc

_P = 0.5


def _plan_indices(seq_len: int) -> np.ndarray:
    rng = np.random.default_rng(0)
    pa = rng.uniform(0.0, 1.0, size=6)
    idx_list = []
    L = seq_len
    if pa[0] < _P:
        start = int(rng.uniform(0.0, L * 0.1))
        idx = np.arange(start, L, dtype=np.int64)
        idx_list.append(idx)
        L = idx.shape[0]
    if pa[1] < _P:
        end = int(rng.uniform(0.0, L * 0.1))
        idx = np.arange(0, L - end, dtype=np.int64)
        idx_list.append(idx)
        L = idx.shape[0]
    if pa[2] < _P:
        delta = float(np.float16(rng.uniform(0.8, 1.2)))
        idx = np.floor(np.arange(0.0, L, delta)).astype(np.int64)
        idx = np.clip(idx, 0, L - 1)
        idx_list.append(idx)
        L = idx.shape[0]
    if pa[3] < _P:
        margin = int(0.1 * L)
        center = int(rng.uniform(margin, L - margin))
        delta = float(np.float16(rng.uniform(0.5, 1.5)))
        mid = np.arange(center - margin, center + margin, delta).astype(np.int64)
        mid = np.clip(mid, 0, L - 1)
        idx = np.concatenate([
            np.arange(0, center - margin, dtype=np.int64),
            mid,
            np.arange(center + margin, L, dtype=np.int64),
        ])
        idx_list.append(idx)
        L = idx.shape[0]
    if pa[4] < _P:
        margin = int(0.1 * L)
        center = int(rng.uniform(margin, L - margin))
        mid = rng.permutation(np.arange(center - margin, center + margin, dtype=np.int64))
        idx = np.concatenate([
            np.arange(0, center - margin, dtype=np.int64),
            mid,
            np.arange(center + margin, L, dtype=np.int64),
        ])
        idx_list.append(idx)
        L = idx.shape[0]
    if pa[4] < _P:
        margin = int(0.1 * L)
        center = int(rng.uniform(margin, L - margin))
        idx = np.concatenate([
            np.arange(0, center - margin, dtype=np.int64),
            np.arange(center - margin, center + margin, dtype=np.int64)[::-1],
            np.arange(center + margin, L, dtype=np.int64),
        ])
        idx_list.append(idx)
        L = idx.shape[0]
    final = idx_list[0]
    for idx in idx_list[1:]:
        final = final[idx]
    return final.astype(np.int32)


_SEQ_LEN = 4096
_FEAT = 512
_IDX_NP = _plan_indices(_SEQ_LEN)
_OUT_LEN = int(_IDX_NP.shape[0])

_NUM_WORKERS = 32
_PAD_LEN = ((_OUT_LEN + 8 * _NUM_WORKERS - 1) // (8 * _NUM_WORKERS)) * (8 * _NUM_WORKERS)
_ROWS_PER_WORKER = _PAD_LEN // _NUM_WORKERS
assert _ROWS_PER_WORKER <= 128
_IDX_PAD_NP = np.zeros((_PAD_LEN,), dtype=np.int32)
_IDX_PAD_NP[:_OUT_LEN] = _IDX_NP


_CHUNK_OFFS = (0, 56, _ROWS_PER_WORKER)
_N_CHUNKS = len(_CHUNK_OFFS) - 1
assert all(o % 8 == 0 for o in _CHUNK_OFFS)


def _sc_gather(a_hbm, b_hbm, idx_hbm, out_a_hbm, out_b_hbm,
               idx_v, rows_a, rows_b, gsems, wsems):
    info = plsc.get_sparse_core_info()
    wid = lax.axis_index("s") * info.num_cores + lax.axis_index("c")
    base = wid * _ROWS_PER_WORKER
    pltpu.sync_copy(idx_hbm.at[pl.ds(base, _ROWS_PER_WORKER)], idx_v)
    gathers = []
    for c in range(_N_CHUNKS):
        off, n = _CHUNK_OFFS[c], _CHUNK_OFFS[c + 1] - _CHUNK_OFFS[c]
        sl = pl.ds(off, n)
        for j, (src, buf) in enumerate(((a_hbm, rows_a), (b_hbm, rows_b))):
            cp = pltpu.async_copy(src.at[idx_v.at[sl]], buf.at[sl],
                                  gsems.at[c * 2 + j])
            gathers.append((cp, buf, off, n))
    writes = []
    for k, (cp, buf, off, n) in enumerate(gathers):
        cp.wait()
        dst = out_a_hbm if k % 2 == 0 else out_b_hbm
        writes.append(pltpu.async_copy(buf.at[pl.ds(off, n)],
                                       dst.at[pl.ds(base + off, n)],
                                       wsems.at[k]))
    for w in writes:
        w.wait()


@jax.jit
def kernel(seq_a, seq_b):
    idx = jnp.asarray(_IDX_PAD_NP)
    mesh = plsc.VectorSubcoreMesh(core_axis_name="c", subcore_axis_name="s")
    out_a, out_b = pl.kernel(
        _sc_gather,
        mesh=mesh,
        out_type=(
            jax.ShapeDtypeStruct((_PAD_LEN, _FEAT), jnp.float32),
            jax.ShapeDtypeStruct((_PAD_LEN, _FEAT), jnp.float32),
        ),
        scratch_types=[
            pltpu.VMEM((_ROWS_PER_WORKER,), jnp.int32),
            pltpu.VMEM((_ROWS_PER_WORKER, _FEAT), jnp.float32),
            pltpu.VMEM((_ROWS_PER_WORKER, _FEAT), jnp.float32),
            pltpu.SemaphoreType.DMA((_N_CHUNKS * 2,)),
            pltpu.SemaphoreType.DMA((_N_CHUNKS * 2,)),
        ],
    )(seq_a, seq_b, idx)
    return jnp.stack([out_a[:_OUT_LEN], out_b[:_OUT_LEN]], axis=0)


# speedup vs baseline: 4.0366x; 1.8527x over previous
import functools

import jax
import jax.numpy as jnp
import numpy as np
from jax import lax
from jax.experimental import pallas as pl
from jax.experimental.pallas import tpu as pltpu
from jax.experimental.pallas import tpu_sc as plsc

_P = 0.5


def _plan_indices(seq_len: int) -> np.ndarray:
    rng = np.random.default_rng(0)
    pa = rng.uniform(0.0, 1.0, size=6)
    idx_list = []
    L = seq_len
    if pa[0] < _P:
        start = int(rng.uniform(0.0, L * 0.1))
        idx = np.arange(start, L, dtype=np.int64)
        idx_list.append(idx)
        L = idx.shape[0]
    if pa[1] < _P:
        end = int(rng.uniform(0.0, L * 0.1))
        idx = np.arange(0, L - end, dtype=np.int64)
        idx_list.append(idx)
        L = idx.shape[0]
    if pa[2] < _P:
        delta = float(np.float16(rng.uniform(0.8, 1.2)))
        idx = np.floor(np.arange(0.0, L, delta)).astype(np.int64)
        idx = np.clip(idx, 0, L - 1)
        idx_list.append(idx)
        L = idx.shape[0]
    if pa[3] < _P:
        margin = int(0.1 * L)
        center = int(rng.uniform(margin, L - margin))
        delta = float(np.float16(rng.uniform(0.5, 1.5)))
        mid = np.arange(center - margin, center + margin, delta).astype(np.int64)
        mid = np.clip(mid, 0, L - 1)
        idx = np.concatenate([
            np.arange(0, center - margin, dtype=np.int64),
            mid,
            np.arange(center + margin, L, dtype=np.int64),
        ])
        idx_list.append(idx)
        L = idx.shape[0]
    if pa[4] < _P:
        margin = int(0.1 * L)
        center = int(rng.uniform(margin, L - margin))
        mid = rng.permutation(np.arange(center - margin, center + margin, dtype=np.int64))
        idx = np.concatenate([
            np.arange(0, center - margin, dtype=np.int64),
            mid,
            np.arange(center + margin, L, dtype=np.int64),
        ])
        idx_list.append(idx)
        L = idx.shape[0]
    if pa[4] < _P:
        margin = int(0.1 * L)
        center = int(rng.uniform(margin, L - margin))
        idx = np.concatenate([
            np.arange(0, center - margin, dtype=np.int64),
            np.arange(center - margin, center + margin, dtype=np.int64)[::-1],
            np.arange(center + margin, L, dtype=np.int64),
        ])
        idx_list.append(idx)
        L = idx.shape[0]
    final = idx_list[0]
    for idx in idx_list[1:]:
        final = final[idx]
    return final.astype(np.int32)


_SEQ_LEN = 4096
_FEAT = 512
_IDX_NP = _plan_indices(_SEQ_LEN)
_OUT_LEN = int(_IDX_NP.shape[0])

_NUM_WORKERS = 32
_ROWS_PER_WORKER = -(-_OUT_LEN // (8 * _NUM_WORKERS)) * 8
assert _ROWS_PER_WORKER <= 128
_LAST_BASE = _OUT_LEN - _ROWS_PER_WORKER
assert _LAST_BASE % 8 == 0 and _OUT_LEN % 8 == 0

_CHUNK_OFFS = (0, 56, _ROWS_PER_WORKER)
_N_CHUNKS = len(_CHUNK_OFFS) - 1
assert all(o % 8 == 0 for o in _CHUNK_OFFS)


def _sc_gather(a_hbm, b_hbm, idx_hbm, out_hbm,
               idx_v, rows_a, rows_b, gsems, wsems):
    info = plsc.get_sparse_core_info()
    wid = lax.axis_index("s") * info.num_cores + lax.axis_index("c")
    base = jnp.where(wid == _NUM_WORKERS - 1, _LAST_BASE,
                     wid * _ROWS_PER_WORKER)
    pltpu.sync_copy(idx_hbm.at[pl.ds(base, _ROWS_PER_WORKER)], idx_v)
    gathers = []
    for c in range(_N_CHUNKS):
        off, n = _CHUNK_OFFS[c], _CHUNK_OFFS[c + 1] - _CHUNK_OFFS[c]
        sl = pl.ds(off, n)
        for j, (src, buf) in enumerate(((a_hbm, rows_a), (b_hbm, rows_b))):
            cp = pltpu.async_copy(src.at[idx_v.at[sl]], buf.at[sl],
                                  gsems.at[c * 2 + j])
            gathers.append((cp, buf, off, n, j))
    writes = []
    for k, (cp, buf, off, n, j) in enumerate(gathers):
        cp.wait()
        writes.append(pltpu.async_copy(buf.at[pl.ds(off, n)],
                                       out_hbm.at[pl.ds(base + off + j * _OUT_LEN, n)],
                                       wsems.at[k]))
    for w in writes:
        w.wait()


@jax.jit
def kernel(seq_a, seq_b):
    idx = jnp.asarray(_IDX_NP)
    mesh = plsc.VectorSubcoreMesh(core_axis_name="c", subcore_axis_name="s")
    out = pl.kernel(
        _sc_gather,
        mesh=mesh,
        out_type=jax.ShapeDtypeStruct((2 * _OUT_LEN, _FEAT), jnp.float32),
        scratch_types=[
            pltpu.VMEM((_ROWS_PER_WORKER,), jnp.int32),
            pltpu.VMEM((_ROWS_PER_WORKER, _FEAT), jnp.float32),
            pltpu.VMEM((_ROWS_PER_WORKER, _FEAT), jnp.float32),
            pltpu.SemaphoreType.DMA((_N_CHUNKS * 2,)),
            pltpu.SemaphoreType.DMA((_N_CHUNKS * 2,)),
        ],
    )(seq_a, seq_b, idx)
    return out.reshape(2, _OUT_LEN, _FEAT)
